# baseline (speedup 1.0000x reference)
import functools

import jax
import jax.numpy as jnp
from jax import lax
from jax.experimental import pallas as pl
from jax.experimental.pallas import tpu as pltpu
from jax.experimental.pallas import tpu_sc as plsc


def _scores_body(x_ref, w_ref, b_ref, tce_ref, id_ref, vals_ref):
    e = jnp.dot(x_ref[...], w_ref[...], preferred_element_type=jnp.float32)
    e = e + b_ref[...]
    s = jnp.sum(tce_ref[...] * e[:, None, :], axis=2)
    m = jnp.max(s, axis=1, keepdims=True)
    p = jnp.exp(s - m)
    p = p / jnp.sum(p, axis=1, keepdims=True)
    ids = id_ref[...]
    eq = ids[:, :, None] == ids[:, None, :]
    vals_ref[...] = jnp.sum(jnp.where(eq, p[:, None, :], 0.0), axis=2) + 1e-6


def _tc_scores(x, W, b2, tce, tid, block_b):
    B, D = x.shape
    _, NC, EC = tce.shape
    grid = (B // block_b,)
    return pl.pallas_call(
        _scores_body,
        grid=grid,
        in_specs=[
            pl.BlockSpec((block_b, D), lambda i: (i, 0)),
            pl.BlockSpec((D, EC), lambda i: (0, 0)),
            pl.BlockSpec((1, EC), lambda i: (0, 0)),
            pl.BlockSpec((block_b, NC, EC), lambda i: (i, 0, 0)),
            pl.BlockSpec((block_b, NC), lambda i: (i, 0)),
        ],
        out_specs=pl.BlockSpec((block_b, NC), lambda i: (i, 0)),
        out_shape=jax.ShapeDtypeStruct((B, NC), jnp.float32),
    )(x, W, b2, tce, tid)


def _make_sc_fill_scatter(B, V, n_chunks, chunk_w):
    info = plsc.get_sparse_core_info()
    n_cores, n_sub = info.num_cores, info.num_subcores
    nw = n_cores * n_sub
    rows_per_w = B // nw
    chunks_per_w = n_chunks // nw
    mesh = plsc.VectorSubcoreMesh(core_axis_name="c", subcore_axis_name="s")

    @functools.partial(
        pl.kernel,
        out_type=jax.ShapeDtypeStruct((B * V,), jnp.float32),
        mesh=mesh,
        scratch_types=[
            pltpu.VMEM((chunks_per_w, chunk_w), jnp.int32),
            pltpu.VMEM((chunks_per_w, chunk_w), jnp.float32),
            pltpu.VMEM((V,), jnp.float32),
            pltpu.SemaphoreType.DMA,
            pltpu.SemaphoreType.DMA,
        ],
    )
    def sc_kernel(idx_hbm, val_hbm, out_hbm, idx_v, val_v, rowbuf, sem_f, sem_s):
        wid = lax.axis_index("s") * n_cores + lax.axis_index("c")

        def fill(i, carry):
            rowbuf[pl.ds(i * 16, 16)] = jnp.full((16,), 1e-6, jnp.float32)
            return carry

        lax.fori_loop(0, V // 16, fill, 0)

        pltpu.sync_copy(idx_hbm.at[pl.ds(wid * chunks_per_w, chunks_per_w)], idx_v)
        pltpu.sync_copy(val_hbm.at[pl.ds(wid * chunks_per_w, chunks_per_w)], val_v)

        fills = [
            pltpu.async_copy(
                rowbuf, out_hbm.at[pl.ds((wid * rows_per_w + r) * V, V)], sem_f
            )
            for r in range(rows_per_w)
        ]
        for f in fills:
            f.wait()

        scats = [
            pltpu.async_copy(val_v.at[c], out_hbm.at[idx_v.at[c]], sem_s)
            for c in range(chunks_per_w)
        ]
        for s in scats:
            s.wait()

    return sc_kernel


def kernel(x, top_city_emb, top_city_id, prob, W, b):
    B, D = x.shape
    _, NC, EC = top_city_emb.shape
    V = prob.shape[1]

    vals = _tc_scores(x, W, b.reshape(1, EC), top_city_emb, top_city_id, 64)

    flat_idx = jnp.arange(B, dtype=jnp.int32)[:, None] * V + top_city_id
    chunk_w = 128
    idx2 = flat_idx.reshape(B // 2, 2 * NC)
    val2 = vals.reshape(B // 2, 2 * NC)
    pad = chunk_w - 2 * NC
    idx_c = jnp.concatenate([idx2, idx2[:, :pad]], axis=1)
    val_c = jnp.concatenate([val2, val2[:, :pad]], axis=1)

    sc = _make_sc_fill_scatter(B, V, B // 2, chunk_w)
    out_flat = sc(idx_c, val_c)
    return out_flat.reshape(B, V)


# speedup vs baseline: 1.1325x; 1.1325x over previous
import functools

import jax
import jax.numpy as jnp
from jax import lax
from jax.experimental import pallas as pl
from jax.experimental.pallas import tpu as pltpu
from jax.experimental.pallas import tpu_sc as plsc


def _scores_body(wt_ref, xt_ref, b_ref, tce_ref, id_ref, vals_ref):
    eT = jnp.dot(wt_ref[...], xt_ref[...], preferred_element_type=jnp.float32)
    eT = eT + b_ref[...]
    s = jnp.sum(tce_ref[...] * eT[None, :, :], axis=1)
    m = jnp.max(s, axis=0, keepdims=True)
    p = jnp.exp(s - m)
    p = p / jnp.sum(p, axis=0, keepdims=True)
    ids = id_ref[...]
    eq = ids[:, None, :] == ids[None, :, :]
    vals_ref[...] = jnp.sum(jnp.where(eq, p[None, :, :], 0.0), axis=1) + 1e-6


def _tc_scores_t(xt, wt, b_col, tce_t, tid_t, block_b):
    D, B = xt.shape
    NC, EC, _ = tce_t.shape
    return pl.pallas_call(
        _scores_body,
        grid=(B // block_b,),
        in_specs=[
            pl.BlockSpec((EC, D), lambda i: (0, 0)),
            pl.BlockSpec((D, block_b), lambda i: (0, i)),
            pl.BlockSpec((EC, 1), lambda i: (0, 0)),
            pl.BlockSpec((NC, EC, block_b), lambda i: (0, 0, i)),
            pl.BlockSpec((NC, block_b), lambda i: (0, i)),
        ],
        out_specs=pl.BlockSpec((NC, block_b), lambda i: (0, i)),
        out_shape=jax.ShapeDtypeStruct((NC, B), jnp.float32),
    )(wt, xt, b_col, tce_t, tid_t)


def _make_sc_fill_scatter(B, V, n_chunks, chunk_w):
    info = plsc.get_sparse_core_info()
    n_cores, n_sub = info.num_cores, info.num_subcores
    nw = n_cores * n_sub
    rows_per_w = B // nw
    chunks_per_w = n_chunks // nw
    mesh = plsc.VectorSubcoreMesh(core_axis_name="c", subcore_axis_name="s")

    @functools.partial(
        pl.kernel,
        out_type=jax.ShapeDtypeStruct((B * V,), jnp.float32),
        mesh=mesh,
        scratch_types=[
            pltpu.VMEM((chunks_per_w, chunk_w), jnp.int32),
            pltpu.VMEM((chunks_per_w, chunk_w), jnp.float32),
            pltpu.VMEM((V,), jnp.float32),
            pltpu.SemaphoreType.DMA,
            pltpu.SemaphoreType.DMA,
        ],
    )
    def sc_kernel(idx_hbm, val_hbm, out_hbm, idx_v, val_v, rowbuf, sem_f, sem_s):
        wid = lax.axis_index("s") * n_cores + lax.axis_index("c")

        def fill(i, carry):
            rowbuf[pl.ds(i * 16, 16)] = jnp.full((16,), 1e-6, jnp.float32)
            return carry

        lax.fori_loop(0, V // 16, fill, 0)

        pltpu.sync_copy(idx_hbm.at[pl.ds(wid * chunks_per_w, chunks_per_w)], idx_v)
        pltpu.sync_copy(val_hbm.at[pl.ds(wid * chunks_per_w, chunks_per_w)], val_v)

        fills = [
            pltpu.async_copy(
                rowbuf, out_hbm.at[pl.ds((wid * rows_per_w + r) * V, V)], sem_f
            )
            for r in range(rows_per_w)
        ]
        for f in fills:
            f.wait()

        scats = [
            pltpu.async_copy(val_v.at[c], out_hbm.at[idx_v.at[c]], sem_s)
            for c in range(chunks_per_w)
        ]
        for s in scats:
            s.wait()

    return sc_kernel


def kernel(x, top_city_emb, top_city_id, prob, W, b):
    B, D = x.shape
    _, NC, EC = top_city_emb.shape
    V = prob.shape[1]

    tce_t = jnp.transpose(top_city_emb, (1, 2, 0))
    tid_t = top_city_id.T
    wt = W.T
    xt = x.T

    vals_t = _tc_scores_t(xt, wt, b.reshape(EC, 1), tce_t, tid_t, 128)
    vals = vals_t.T

    flat_idx = jnp.arange(B, dtype=jnp.int32)[:, None] * V + top_city_id
    chunk_w = 128
    idx2 = flat_idx.reshape(B // 2, 2 * NC)
    val2 = vals.reshape(B // 2, 2 * NC)
    pad = chunk_w - 2 * NC
    idx_c = jnp.concatenate([idx2, idx2[:, :pad]], axis=1)
    val_c = jnp.concatenate([val2, val2[:, :pad]], axis=1)

    sc = _make_sc_fill_scatter(B, V, B // 2, chunk_w)
    out_flat = sc(idx_c, val_c)
    return out_flat.reshape(B, V)


# speedup vs baseline: 1.4343x; 1.2665x over previous
import functools

import jax
import jax.numpy as jnp
from jax import lax
from jax.experimental import pallas as pl
from jax.experimental.pallas import tpu as pltpu
from jax.experimental.pallas import tpu_sc as plsc


def _scores_body(wt_ref, xt_ref, b_ref, tce_ref, id_ref, vals_ref):
    eT = jnp.dot(wt_ref[...], xt_ref[...], preferred_element_type=jnp.float32)
    eT = eT + b_ref[...]
    s = jnp.sum(tce_ref[...] * eT[None, :, :], axis=1)
    m = jnp.max(s, axis=0, keepdims=True)
    p = jnp.exp(s - m)
    p = p / jnp.sum(p, axis=0, keepdims=True)
    ids = id_ref[...]
    eq = ids[:, None, :] == ids[None, :, :]
    vals_ref[...] = jnp.sum(jnp.where(eq, p[None, :, :], 0.0), axis=1) + 1e-6


def _tc_scores_t(xt, wt, b_col, tce_t, tid_t, block_b):
    D, B = xt.shape
    NC, EC, _ = tce_t.shape
    return pl.pallas_call(
        _scores_body,
        grid=(B // block_b,),
        in_specs=[
            pl.BlockSpec((EC, D), lambda i: (0, 0)),
            pl.BlockSpec((D, block_b), lambda i: (0, i)),
            pl.BlockSpec((EC, 1), lambda i: (0, 0)),
            pl.BlockSpec((NC, EC, block_b), lambda i: (0, 0, i)),
            pl.BlockSpec((NC, block_b), lambda i: (0, i)),
        ],
        out_specs=pl.BlockSpec((NC, block_b), lambda i: (0, i)),
        out_shape=jax.ShapeDtypeStruct((NC, B), jnp.float32),
    )(wt, xt, b_col, tce_t, tid_t)


def _make_sc_fill_scatter(B, V, RS, n_chunks, chunk_w):
    info = plsc.get_sparse_core_info()
    n_cores, n_sub = info.num_cores, info.num_subcores
    nw = n_cores * n_sub
    rows_per_w = B // nw
    chunks_per_w = n_chunks // nw
    mesh = plsc.VectorSubcoreMesh(core_axis_name="c", subcore_axis_name="s")

    @functools.partial(
        pl.kernel,
        out_type=jax.ShapeDtypeStruct((B * RS,), jnp.float32),
        mesh=mesh,
        scratch_types=[
            pltpu.VMEM((chunks_per_w, chunk_w), jnp.int32),
            pltpu.VMEM((chunks_per_w, chunk_w), jnp.float32),
            pltpu.VMEM((RS,), jnp.float32),
            pltpu.SemaphoreType.DMA,
            pltpu.SemaphoreType.DMA,
        ],
    )
    def sc_kernel(idx_hbm, val_hbm, out_hbm, idx_v, val_v, rowbuf, sem_f, sem_s):
        wid = lax.axis_index("s") * n_cores + lax.axis_index("c")

        def fill(i, carry):
            rowbuf[pl.ds(i * 16, 16)] = jnp.full((16,), 1e-6, jnp.float32)
            return carry

        lax.fori_loop(0, RS // 16, fill, 0)

        pltpu.sync_copy(idx_hbm.at[pl.ds(wid * chunks_per_w, chunks_per_w)], idx_v)
        pltpu.sync_copy(val_hbm.at[pl.ds(wid * chunks_per_w, chunks_per_w)], val_v)

        fills = [
            pltpu.async_copy(
                rowbuf, out_hbm.at[pl.ds((wid * rows_per_w + r) * RS, RS)], sem_f
            )
            for r in range(rows_per_w)
        ]
        for f in fills:
            f.wait()

        scats = [
            pltpu.async_copy(val_v.at[c], out_hbm.at[idx_v.at[c]], sem_s)
            for c in range(chunks_per_w)
        ]
        for s in scats:
            s.wait()

    return sc_kernel


def _relayout_body(in_ref, out_ref):
    rows, rs = out_ref.shape[0], in_ref.shape[0] // out_ref.shape[0]
    out_ref[...] = in_ref[...].reshape(rows, rs)[:, : out_ref.shape[1]]


def _tc_relayout(flat, B, V, RS, rows):
    return pl.pallas_call(
        _relayout_body,
        grid=(B // rows,),
        in_specs=[pl.BlockSpec((rows * RS,), lambda i: (i,))],
        out_specs=pl.BlockSpec((rows, V), lambda i: (i, 0)),
        out_shape=jax.ShapeDtypeStruct((B, V), jnp.float32),
    )(flat)


def kernel(x, top_city_emb, top_city_id, prob, W, b):
    B, D = x.shape
    _, NC, EC = top_city_emb.shape
    V = prob.shape[1]

    tce_t = jnp.transpose(top_city_emb, (1, 2, 0))
    tid_t = top_city_id.T
    wt = W.T
    xt = x.T

    vals_t = _tc_scores_t(xt, wt, b.reshape(EC, 1), tce_t, tid_t, 128)
    vals = vals_t.T

    RS = ((V + 1023) // 1024) * 1024
    flat_idx = jnp.arange(B, dtype=jnp.int32)[:, None] * RS + top_city_id
    chunk_w = 128
    idx2 = flat_idx.reshape(B // 2, 2 * NC)
    val2 = vals.reshape(B // 2, 2 * NC)
    pad = chunk_w - 2 * NC
    idx_c = jnp.concatenate([idx2, idx2[:, :pad]], axis=1)
    val_c = jnp.concatenate([val2, val2[:, :pad]], axis=1)

    sc = _make_sc_fill_scatter(B, V, RS, B // 2, chunk_w)
    out_flat = sc(idx_c, val_c)
    return _tc_relayout(out_flat, B, V, RS, 8)


# speedup vs baseline: 1.4379x; 1.0025x over previous
import functools

import jax
import jax.numpy as jnp
from jax import lax
from jax.experimental import pallas as pl
from jax.experimental.pallas import tpu as pltpu
from jax.experimental.pallas import tpu_sc as plsc


def _scores_body(wt_ref, xt_ref, b_ref, tce_ref, id_ref, vals_ref):
    eT = jnp.dot(wt_ref[...], xt_ref[...], preferred_element_type=jnp.float32)
    eT = eT + b_ref[...]
    s = jnp.sum(tce_ref[...] * eT[None, :, :], axis=1)
    m = jnp.max(s, axis=0, keepdims=True)
    p = jnp.exp(s - m)
    p = p / jnp.sum(p, axis=0, keepdims=True)
    ids = id_ref[...]
    eq = ids[:, None, :] == ids[None, :, :]
    vals_ref[...] = jnp.sum(jnp.where(eq, p[None, :, :], 0.0), axis=1) + 1e-6


def _tc_scores_t(xt, wt, b_col, tce_t, tid_t, block_b):
    D, B = xt.shape
    NC, EC, _ = tce_t.shape
    return pl.pallas_call(
        _scores_body,
        grid=(B // block_b,),
        in_specs=[
            pl.BlockSpec((EC, D), lambda i: (0, 0)),
            pl.BlockSpec((D, block_b), lambda i: (0, i)),
            pl.BlockSpec((EC, 1), lambda i: (0, 0)),
            pl.BlockSpec((NC, EC, block_b), lambda i: (0, 0, i)),
            pl.BlockSpec((NC, block_b), lambda i: (0, i)),
        ],
        out_specs=pl.BlockSpec((NC, block_b), lambda i: (0, i)),
        out_shape=jax.ShapeDtypeStruct((NC, B), jnp.float32),
    )(wt, xt, b_col, tce_t, tid_t)


def _make_sc_fill_scatter(B, V, RS, n_chunks, chunk_w):
    info = plsc.get_sparse_core_info()
    n_cores, n_sub = info.num_cores, info.num_subcores
    nw = n_cores * n_sub
    rows_per_w = B // nw
    chunks_per_w = n_chunks // nw
    mesh = plsc.VectorSubcoreMesh(core_axis_name="c", subcore_axis_name="s")

    @functools.partial(
        pl.kernel,
        out_type=jax.ShapeDtypeStruct((B * RS,), jnp.float32),
        mesh=mesh,
        scratch_types=[
            pltpu.VMEM((chunks_per_w, chunk_w), jnp.int32),
            pltpu.VMEM((chunks_per_w, chunk_w), jnp.float32),
            pltpu.VMEM((RS,), jnp.float32),
            pltpu.SemaphoreType.DMA,
            pltpu.SemaphoreType.DMA,
        ],
    )
    def sc_kernel(idx_hbm, val_hbm, out_hbm, idx_v, val_v, rowbuf, sem_f, sem_s):
        wid = lax.axis_index("s") * n_cores + lax.axis_index("c")

        def fill(i, carry):
            rowbuf[pl.ds(i * 16, 16)] = jnp.full((16,), 1e-6, jnp.float32)
            return carry

        lax.fori_loop(0, RS // 16, fill, 0)

        pltpu.sync_copy(idx_hbm.at[pl.ds(wid * chunks_per_w, chunks_per_w)], idx_v)
        pltpu.sync_copy(val_hbm.at[pl.ds(wid * chunks_per_w, chunks_per_w)], val_v)

        fills = [
            pltpu.async_copy(
                rowbuf, out_hbm.at[pl.ds((wid * rows_per_w + r) * RS, RS)], sem_f
            )
            for r in range(rows_per_w)
        ]
        for f in fills:
            f.wait()

        scats = [
            pltpu.async_copy(val_v.at[c], out_hbm.at[idx_v.at[c]], sem_s)
            for c in range(chunks_per_w)
        ]
        for s in scats:
            s.wait()

    return sc_kernel


def _relayout_body(in_ref, out_ref):
    rows, rs = out_ref.shape[0], in_ref.shape[0] // out_ref.shape[0]
    out_ref[...] = in_ref[...].reshape(rows, rs)[:, : out_ref.shape[1]]


def _tc_relayout(flat, B, V, RS, rows):
    return pl.pallas_call(
        _relayout_body,
        grid=(B // rows,),
        in_specs=[pl.BlockSpec((rows * RS,), lambda i: (i,))],
        out_specs=pl.BlockSpec((rows, V), lambda i: (i, 0)),
        out_shape=jax.ShapeDtypeStruct((B, V), jnp.float32),
    )(flat)


def kernel(x, top_city_emb, top_city_id, prob, W, b):
    B, D = x.shape
    _, NC, EC = top_city_emb.shape
    V = prob.shape[1]

    tce_t = jnp.transpose(top_city_emb, (1, 2, 0))
    tid_t = top_city_id.T
    wt = W.T
    xt = x.T

    vals_t = _tc_scores_t(xt, wt, b.reshape(EC, 1), tce_t, tid_t, 128)
    vals = vals_t.T

    RS = ((V + 1023) // 1024) * 1024
    flat_idx = jnp.arange(B, dtype=jnp.int32)[:, None] * RS + top_city_id
    chunk_w = 128
    idx2 = flat_idx.reshape(B // 2, 2 * NC)
    val2 = vals.reshape(B // 2, 2 * NC)
    pad = chunk_w - 2 * NC
    idx_c = jnp.concatenate([idx2, idx2[:, :pad]], axis=1)
    val_c = jnp.concatenate([val2, val2[:, :pad]], axis=1)

    sc = _make_sc_fill_scatter(B, V, RS, B // 2, chunk_w)
    out_flat = sc(idx_c, val_c)
    return _tc_relayout(out_flat, B, V, RS, 16)
